# baseline (speedup 1.0000x reference)
import dataclasses
import functools

import jax
import jax.numpy as jnp
from jax import lax
from jax.experimental import pallas as pl
from jax.experimental.pallas import tpu as pltpu
from jax.experimental.pallas import tpu_sc as plsc

_N = 10000
_E = 320000
_G = 64
_NC = 2
_NS = 16
_NW = _NC * _NS
_EPW = _E // _NW
_CH = 80
_NCHUNK = _EPW // _CH
_NPAD = 10240
_RPS = _NPAD // _NS
_ZR = 128
_BLK = 400
_NBLK = _N // _BLK


def _vector_mesh():
    return plsc.VectorSubcoreMesh(core_axis_name="c", subcore_axis_name="s")


def _sc_params():
    cp = pltpu.CompilerParams()
    if "needs_layout_passes" in pltpu.CompilerParams.__dataclass_fields__:
        cp = dataclasses.replace(cp, needs_layout_passes=False)
    return cp


def _sc_degree(col, ew, zeros):
    @functools.partial(
        pl.kernel,
        out_type=jax.ShapeDtypeStruct((_NC, _NPAD, 128), jnp.float32),
        mesh=_vector_mesh(),
        compiler_params=_sc_params(),
        scratch_types=[
            pltpu.VMEM((_CH,), jnp.int32),
            pltpu.VMEM((_CH,), jnp.float32),
            pltpu.VMEM((_CH, 128), jnp.float32),
            pltpu.VMEM_SHARED((_NPAD, 128), jnp.float32),
        ],
    )
    def deg_kernel(col_hbm, ew_hbm, zero_hbm, out_hbm, colv, ewv, srcv, acc):
        cid = lax.axis_index("c")
        sid = lax.axis_index("s")
        wid = sid * _NC + cid
        zero16 = jnp.zeros((16,), jnp.float32)

        @pl.loop(0, _CH)
        def _(r):
            for kk in range(8):
                srcv[r, pl.ds(kk * 16, 16)] = zero16

        for j in range(_RPS // _ZR):
            pltpu.sync_copy(zero_hbm, acc.at[pl.ds(sid * _RPS + j * _ZR, _ZR)])
        plsc.subcore_barrier()

        @pl.loop(0, _NCHUNK)
        def _(ch):
            base = wid * _EPW + ch * _CH
            pltpu.sync_copy(col_hbm.at[pl.ds(base, _CH)], colv)
            pltpu.sync_copy(ew_hbm.at[pl.ds(base, _CH)], ewv)

            @pl.loop(0, _CH)
            def _(e):
                w = plsc.load_gather(ewv, [jnp.zeros((16,), jnp.int32) + e])
                srcv[e, pl.ds(0, 16)] = w

            pltpu.sync_copy(srcv, acc.at[colv], add=True)

        plsc.subcore_barrier()
        for j in range(_RPS // _ZR):
            off = sid * _RPS + j * _ZR
            pltpu.sync_copy(acc.at[pl.ds(off, _ZR)], out_hbm.at[cid, pl.ds(off, _ZR)])

    return deg_kernel(col, ew, zeros)


_EPWP = 10240
_CCH = 128
_CNCH = _EPWP // _CCH


def _sc_conv(table, row3, col3, ew3, zeros, d):
    @functools.partial(
        pl.kernel,
        out_type=jax.ShapeDtypeStruct((_NC, _NPAD, d), jnp.float32),
        mesh=_vector_mesh(),
        compiler_params=_sc_params(),
        scratch_types=[
            pltpu.VMEM((4, 1, _CCH), jnp.int32),
            pltpu.VMEM((4, 1, _CCH), jnp.int32),
            pltpu.VMEM((4, 1, _CCH), jnp.float32),
            pltpu.VMEM((_CCH, d), jnp.float32),
            pltpu.VMEM((_CCH, d), jnp.float32),
            pltpu.VMEM_SHARED((_NPAD, d), jnp.float32),
            pltpu.SemaphoreType.DMA,
            pltpu.SemaphoreType.DMA,
            pltpu.SemaphoreType.DMA,
            pltpu.SemaphoreType.DMA,
            pltpu.SemaphoreType.DMA,
            pltpu.SemaphoreType.DMA,
            pltpu.SemaphoreType.DMA,
            pltpu.SemaphoreType.DMA,
        ],
    )
    def conv_kernel(table_hbm, row_hbm, col_hbm, ew_hbm, zero_hbm, out_hbm,
                    rowq, colq, ewq, msg0, msg1, acc,
                    g0, g1, s0, s1, i0, i1, i2, i3):
        cid = lax.axis_index("c")
        sid = lax.axis_index("s")
        wid = sid * _NC + cid
        msgs = (msg0, msg1)
        gsems = (g0, g1)
        ssems = (s0, s1)
        isems = (i0, i1, i2, i3)

        for j in range(_RPS // _ZR):
            pltpu.sync_copy(zero_hbm, acc.at[pl.ds(sid * _RPS + j * _ZR, _ZR)])
        plsc.subcore_barrier()

        def idx_start(ch, q):
            pltpu.async_copy(row_hbm.at[wid, ch], rowq.at[q], isems[q])
            pltpu.async_copy(col_hbm.at[wid, ch], colq.at[q], isems[q])
            pltpu.async_copy(ew_hbm.at[wid, ch], ewq.at[q], isems[q])

        def idx_wait(q):
            pltpu.make_async_copy(row_hbm.at[wid, 0], rowq.at[q], isems[q]).wait()
            pltpu.make_async_copy(col_hbm.at[wid, 0], colq.at[q], isems[q]).wait()
            pltpu.make_async_copy(ew_hbm.at[wid, 0], ewq.at[q], isems[q]).wait()

        def scale(b, q):
            zi = jnp.zeros((16,), jnp.int32)

            @pl.loop(0, _CCH)
            def _(e):
                w = plsc.load_gather(ewq, [zi + q, zi, zi + e])
                for kk in range(d // 16):
                    msgs[b][e, pl.ds(kk * 16, 16)] = (
                        msgs[b][e, pl.ds(kk * 16, 16)] * w)

        def gather_start(b, q):
            pltpu.async_copy(table_hbm.at[rowq.at[q, 0]], msgs[b], gsems[b])

        def gather_wait(b):
            pltpu.make_async_copy(table_hbm.at[rowq.at[0, 0]], msgs[b],
                                  gsems[b]).wait()

        def scatter_start(b, q):
            pltpu.async_copy(msgs[b], acc.at[colq.at[q, 0]], ssems[b], add=True)

        def scatter_wait(b):
            pltpu.make_async_copy(msgs[b], acc.at[colq.at[0, 0]], ssems[b]).wait()

        for q in range(4):
            idx_start(q, q)
        idx_wait(0)
        gather_start(0, 0)
        idx_wait(1)
        gather_start(1, 1)

        @pl.loop(0, _CNCH - 4, step=4)
        def _(p):
            gather_wait(0)
            scale(0, 0)
            scatter_start(0, 0)
            gather_wait(1)
            scale(1, 1)
            scatter_start(1, 1)
            scatter_wait(0)
            idx_start(p + 4, 0)
            idx_wait(2)
            gather_start(0, 2)
            scatter_wait(1)
            idx_start(p + 5, 1)
            idx_wait(3)
            gather_start(1, 3)
            gather_wait(0)
            scale(0, 2)
            scatter_start(0, 2)
            gather_wait(1)
            scale(1, 3)
            scatter_start(1, 3)
            scatter_wait(0)
            idx_start(p + 6, 2)
            idx_wait(0)
            gather_start(0, 0)
            scatter_wait(1)
            idx_start(p + 7, 3)
            idx_wait(1)
            gather_start(1, 1)

        gather_wait(0)
        scale(0, 0)
        scatter_start(0, 0)
        gather_wait(1)
        scale(1, 1)
        scatter_start(1, 1)
        scatter_wait(0)
        idx_wait(2)
        gather_start(0, 2)
        scatter_wait(1)
        idx_wait(3)
        gather_start(1, 3)
        gather_wait(0)
        scale(0, 2)
        scatter_start(0, 2)
        gather_wait(1)
        scale(1, 3)
        scatter_start(1, 3)
        scatter_wait(0)
        scatter_wait(1)

        plsc.subcore_barrier()
        for j in range(_RPS // _ZR):
            off = sid * _RPS + j * _ZR
            pltpu.sync_copy(acc.at[pl.ds(off, _ZR)], out_hbm.at[cid, pl.ds(off, _ZR)])

    return conv_kernel(table, row3, col3, ew3, zeros)


def _dinv_block(d0_ref, d1_ref):
    deg = d0_ref[:, 0:1] + d1_ref[:, 0:1] + 1.0
    return lax.rsqrt(deg)


def _tc_stage1(x, d0, d1, w1):
    def body(x_ref, d0_ref, d1_ref, w_ref, o_ref):
        dinv = _dinv_block(d0_ref, d1_ref)
        xw = jnp.dot(x_ref[...], w_ref[...], preferred_element_type=jnp.float32)
        o_ref[...] = xw * dinv

    return pl.pallas_call(
        body,
        grid=(_NBLK,),
        in_specs=[
            pl.BlockSpec((_BLK, 128), lambda i: (i, 0)),
            pl.BlockSpec((_BLK, 128), lambda i: (i, 0)),
            pl.BlockSpec((_BLK, 128), lambda i: (i, 0)),
            pl.BlockSpec((128, 128), lambda i: (0, 0)),
        ],
        out_specs=pl.BlockSpec((_BLK, 128), lambda i: (i, 0)),
        out_shape=jax.ShapeDtypeStruct((_N, 128), jnp.float32),
    )(x, d0, d1, w1)


def _tc_stage2(a0, a1, t1, d0, d1, b1, w2):
    def body(a0_ref, a1_ref, t1_ref, d0_ref, d1_ref, b_ref, w_ref, o_ref):
        dinv = _dinv_block(d0_ref, d1_ref)
        h = dinv * (a0_ref[...] + a1_ref[...] + t1_ref[...]) + b_ref[...]
        h = jnp.maximum(h, 0.0)
        hw = jnp.dot(h, w_ref[...], preferred_element_type=jnp.float32)
        o_ref[...] = hw * dinv

    return pl.pallas_call(
        body,
        grid=(_NBLK,),
        in_specs=[
            pl.BlockSpec((_BLK, 128), lambda i: (i, 0)),
            pl.BlockSpec((_BLK, 128), lambda i: (i, 0)),
            pl.BlockSpec((_BLK, 128), lambda i: (i, 0)),
            pl.BlockSpec((_BLK, 128), lambda i: (i, 0)),
            pl.BlockSpec((_BLK, 128), lambda i: (i, 0)),
            pl.BlockSpec((1, 128), lambda i: (0, 0)),
            pl.BlockSpec((128, 128), lambda i: (0, 0)),
        ],
        out_specs=pl.BlockSpec((_BLK, 128), lambda i: (i, 0)),
        out_shape=jax.ShapeDtypeStruct((_N, 128), jnp.float32),
    )(a0, a1, t1, d0, d1, b1, w2)


def _tc_final(a0, a1, t2, d0, d1, b2, batch3, wl1, bl1, wl2, bl2):
    def body(a0_ref, a1_ref, t2_ref, d0_ref, d1_ref, b_ref, batch_ref,
             wl1_ref, bl1_ref, wl2_ref, bl2_ref,
             yp_ref, yh_ref, out_ref, pooled_ref):
        i = pl.program_id(0)

        @pl.when(i == 0)
        def _():
            pooled_ref[...] = jnp.zeros_like(pooled_ref)

        dinv = _dinv_block(d0_ref, d1_ref)
        h2 = dinv * (a0_ref[...] + a1_ref[...] + t2_ref[...]) + b_ref[...]
        h2 = jnp.maximum(h2, 0.0)
        b = batch_ref[...][0]
        gids = lax.broadcasted_iota(jnp.int32, (_G, _BLK), 0)
        onehot = (jnp.broadcast_to(b, (_G, _BLK)) == gids).astype(jnp.float32)
        pooled_ref[...] += jnp.dot(onehot, h2, preferred_element_type=jnp.float32)

        @pl.when(i == _NBLK - 1)
        def _():
            p = pooled_ref[...]
            h3 = jnp.dot(p, wl1_ref[...], preferred_element_type=jnp.float32)
            h3 = jnp.maximum(h3 + bl1_ref[...], 0.0)
            o = jnp.dot(h3, wl2_ref[...], preferred_element_type=jnp.float32)
            o = o + bl2_ref[...]
            m = jnp.max(o, axis=1, keepdims=True)
            lse = m + jnp.log(jnp.sum(jnp.exp(o - m), axis=1, keepdims=True))
            yp_ref[...] = o - lse
            yh_ref[...] = (o[:, 1:2] > o[:, 0:1]).astype(jnp.float32)
            out_ref[...] = o

    return pl.pallas_call(
        body,
        grid=(_NBLK,),
        in_specs=[
            pl.BlockSpec((_BLK, 128), lambda i: (i, 0)),
            pl.BlockSpec((_BLK, 128), lambda i: (i, 0)),
            pl.BlockSpec((_BLK, 128), lambda i: (i, 0)),
            pl.BlockSpec((_BLK, 128), lambda i: (i, 0)),
            pl.BlockSpec((_BLK, 128), lambda i: (i, 0)),
            pl.BlockSpec((1, 128), lambda i: (0, 0)),
            pl.BlockSpec((1, 1, _BLK), lambda i: (i, 0, 0)),
            pl.BlockSpec((128, 64), lambda i: (0, 0)),
            pl.BlockSpec((1, 64), lambda i: (0, 0)),
            pl.BlockSpec((64, 2), lambda i: (0, 0)),
            pl.BlockSpec((1, 2), lambda i: (0, 0)),
        ],
        out_specs=[
            pl.BlockSpec((_G, 2), lambda i: (0, 0)),
            pl.BlockSpec((_G, 1), lambda i: (0, 0)),
            pl.BlockSpec((_G, 2), lambda i: (0, 0)),
        ],
        out_shape=[
            jax.ShapeDtypeStruct((_G, 2), jnp.float32),
            jax.ShapeDtypeStruct((_G, 1), jnp.float32),
            jax.ShapeDtypeStruct((_G, 2), jnp.float32),
        ],
        scratch_shapes=[pltpu.VMEM((_G, 128), jnp.float32)],
    )(a0, a1, t2, d0, d1, b2, batch3, wl1, bl1, wl2, bl2)


def kernel(x, edge_index, edge_weight, batch, W1, b1, W2, b2, Wl1, bl1, Wl2, bl2):
    row = edge_index[0]
    col = edge_index[1]

    zeros = jnp.zeros((_ZR, 128), jnp.float32)
    deg16 = _sc_degree(col, edge_weight, zeros)
    d0 = deg16[0]
    d1 = deg16[1]

    w2p = jnp.pad(W2, ((0, 0), (0, 64)))
    b2p = jnp.pad(b2, (0, 64)).reshape(1, 128)
    wl1p = jnp.pad(Wl1, ((0, 64), (0, 0)))

    pad = _EPWP - _EPW
    row3 = jnp.pad(row.reshape(_NW, _EPW), ((0, 0), (0, pad))).reshape(
        _NW, _CNCH, 1, _CCH)
    col3 = jnp.pad(col.reshape(_NW, _EPW), ((0, 0), (0, pad))).reshape(
        _NW, _CNCH, 1, _CCH)
    ew3 = jnp.pad(edge_weight.reshape(_NW, _EPW), ((0, 0), (0, pad))).reshape(
        _NW, _CNCH, 1, _CCH)

    table1 = _tc_stage1(x, d0, d1, W1)
    agg1 = _sc_conv(table1, row3, col3, ew3, zeros, 128)
    table2 = _tc_stage2(agg1[0], agg1[1], table1, d0, d1,
                        b1.reshape(1, 128), w2p)
    agg2 = _sc_conv(table2, row3, col3, ew3, zeros, 128)

    batch3 = batch.reshape(_NBLK, 1, _BLK)
    y_prob, y_hat, out = _tc_final(agg2[0], agg2[1], table2, d0, d1,
                                   b2p, batch3,
                                   wl1p, bl1.reshape(1, 64),
                                   Wl2, bl2.reshape(1, 2))
    return (y_prob, y_hat.reshape(_G), out)


# speedup vs baseline: 1.0547x; 1.0547x over previous
import dataclasses
import functools

import jax
import jax.numpy as jnp
from jax import lax
from jax.experimental import pallas as pl
from jax.experimental.pallas import tpu as pltpu
from jax.experimental.pallas import tpu_sc as plsc

_N = 10000
_E = 320000
_G = 64
_NC = 2
_NS = 16
_NW = _NC * _NS
_EPW = _E // _NW
_CH = 80
_NCHUNK = _EPW // _CH
_NPAD = 10240
_RPS = _NPAD // _NS
_ZR = 128
_BLK = 400
_NBLK = _N // _BLK


def _vector_mesh():
    return plsc.VectorSubcoreMesh(core_axis_name="c", subcore_axis_name="s")


def _sc_params():
    cp = pltpu.CompilerParams()
    if "needs_layout_passes" in pltpu.CompilerParams.__dataclass_fields__:
        cp = dataclasses.replace(cp, needs_layout_passes=False)
    return cp


def _sc_degree(col, ew, zeros):
    @functools.partial(
        pl.kernel,
        out_type=jax.ShapeDtypeStruct((_NC, _NPAD, 128), jnp.float32),
        mesh=_vector_mesh(),
        compiler_params=_sc_params(),
        scratch_types=[
            pltpu.VMEM((_CH,), jnp.int32),
            pltpu.VMEM((_CH,), jnp.float32),
            pltpu.VMEM((_CH, 128), jnp.float32),
            pltpu.VMEM_SHARED((_NPAD, 128), jnp.float32),
        ],
    )
    def deg_kernel(col_hbm, ew_hbm, zero_hbm, out_hbm, colv, ewv, srcv, acc):
        cid = lax.axis_index("c")
        sid = lax.axis_index("s")
        wid = sid * _NC + cid
        zero16 = jnp.zeros((16,), jnp.float32)

        @pl.loop(0, _CH)
        def _(r):
            for kk in range(8):
                srcv[r, pl.ds(kk * 16, 16)] = zero16

        for j in range(_RPS // _ZR):
            pltpu.sync_copy(zero_hbm, acc.at[pl.ds(sid * _RPS + j * _ZR, _ZR)])
        plsc.subcore_barrier()

        @pl.loop(0, _NCHUNK)
        def _(ch):
            base = wid * _EPW + ch * _CH
            pltpu.sync_copy(col_hbm.at[pl.ds(base, _CH)], colv)
            pltpu.sync_copy(ew_hbm.at[pl.ds(base, _CH)], ewv)

            @pl.loop(0, _CH)
            def _(e):
                w = plsc.load_gather(ewv, [jnp.zeros((16,), jnp.int32) + e])
                srcv[e, pl.ds(0, 16)] = w

            pltpu.sync_copy(srcv, acc.at[colv], add=True)

        plsc.subcore_barrier()
        for j in range(_RPS // _ZR):
            off = sid * _RPS + j * _ZR
            pltpu.sync_copy(acc.at[pl.ds(off, _ZR)], out_hbm.at[cid, pl.ds(off, _ZR)])

    return deg_kernel(col, ew, zeros)


_EPWP = 10240
_CCH = 128
_CNCH = _EPWP // _CCH


def _sc_conv(table, row3, col3, ew3, zeros, d):
    @functools.partial(
        pl.kernel,
        out_type=jax.ShapeDtypeStruct((_NC, _NPAD, d), jnp.float32),
        mesh=_vector_mesh(),
        compiler_params=_sc_params(),
        scratch_types=[
            pltpu.VMEM((4, 1, _CCH), jnp.int32),
            pltpu.VMEM((4, 1, _CCH), jnp.int32),
            pltpu.VMEM((4, 1, _CCH), jnp.float32),
            pltpu.VMEM((_CCH, d), jnp.float32),
            pltpu.VMEM((_CCH, d), jnp.float32),
            pltpu.VMEM_SHARED((_NPAD, d), jnp.float32),
            pltpu.SemaphoreType.DMA,
            pltpu.SemaphoreType.DMA,
            pltpu.SemaphoreType.DMA,
            pltpu.SemaphoreType.DMA,
            pltpu.SemaphoreType.DMA,
            pltpu.SemaphoreType.DMA,
            pltpu.SemaphoreType.DMA,
            pltpu.SemaphoreType.DMA,
        ],
    )
    def conv_kernel(table_hbm, row_hbm, col_hbm, ew_hbm, zero_hbm, out_hbm,
                    rowq, colq, ewq, msg0, msg1, acc,
                    g0, g1, s0, s1, i0, i1, i2, i3):
        cid = lax.axis_index("c")
        sid = lax.axis_index("s")
        wid = sid * _NC + cid
        msgs = (msg0, msg1)
        gsems = (g0, g1)
        ssems = (s0, s1)
        isems = (i0, i1, i2, i3)

        for j in range(_RPS // _ZR):
            pltpu.sync_copy(zero_hbm, acc.at[pl.ds(sid * _RPS + j * _ZR, _ZR)])
        plsc.subcore_barrier()

        def idx_start(ch, q):
            pltpu.async_copy(row_hbm.at[wid, ch], rowq.at[q], isems[q])
            pltpu.async_copy(col_hbm.at[wid, ch], colq.at[q], isems[q])
            pltpu.async_copy(ew_hbm.at[wid, ch], ewq.at[q], isems[q])

        def idx_wait(q):
            pltpu.make_async_copy(row_hbm.at[wid, 0], rowq.at[q], isems[q]).wait()
            pltpu.make_async_copy(col_hbm.at[wid, 0], colq.at[q], isems[q]).wait()
            pltpu.make_async_copy(ew_hbm.at[wid, 0], ewq.at[q], isems[q]).wait()

        def scale(b, q):
            zi = jnp.zeros((16,), jnp.int32)

            @plsc.parallel_loop(0, _CCH, unroll=4)
            def _(e):
                w = plsc.load_gather(ewq, [zi + q, zi, zi + e])
                for kk in range(d // 16):
                    msgs[b][e, pl.ds(kk * 16, 16)] = (
                        msgs[b][e, pl.ds(kk * 16, 16)] * w)

        def gather_start(b, q):
            pltpu.async_copy(table_hbm.at[rowq.at[q, 0]], msgs[b], gsems[b])

        def gather_wait(b):
            pltpu.make_async_copy(table_hbm.at[rowq.at[0, 0]], msgs[b],
                                  gsems[b]).wait()

        def scatter_start(b, q):
            pltpu.async_copy(msgs[b], acc.at[colq.at[q, 0]], ssems[b], add=True)

        def scatter_wait(b):
            pltpu.make_async_copy(msgs[b], acc.at[colq.at[0, 0]], ssems[b]).wait()

        for q in range(4):
            idx_start(q, q)
        idx_wait(0)
        gather_start(0, 0)
        idx_wait(1)
        gather_start(1, 1)

        @pl.loop(0, _CNCH - 4, step=4)
        def _(p):
            gather_wait(0)
            scale(0, 0)
            scatter_start(0, 0)
            gather_wait(1)
            scale(1, 1)
            scatter_start(1, 1)
            scatter_wait(0)
            idx_start(p + 4, 0)
            idx_wait(2)
            gather_start(0, 2)
            scatter_wait(1)
            idx_start(p + 5, 1)
            idx_wait(3)
            gather_start(1, 3)
            gather_wait(0)
            scale(0, 2)
            scatter_start(0, 2)
            gather_wait(1)
            scale(1, 3)
            scatter_start(1, 3)
            scatter_wait(0)
            idx_start(p + 6, 2)
            idx_wait(0)
            gather_start(0, 0)
            scatter_wait(1)
            idx_start(p + 7, 3)
            idx_wait(1)
            gather_start(1, 1)

        gather_wait(0)
        scale(0, 0)
        scatter_start(0, 0)
        gather_wait(1)
        scale(1, 1)
        scatter_start(1, 1)
        scatter_wait(0)
        idx_wait(2)
        gather_start(0, 2)
        scatter_wait(1)
        idx_wait(3)
        gather_start(1, 3)
        gather_wait(0)
        scale(0, 2)
        scatter_start(0, 2)
        gather_wait(1)
        scale(1, 3)
        scatter_start(1, 3)
        scatter_wait(0)
        scatter_wait(1)

        plsc.subcore_barrier()
        for j in range(_RPS // _ZR):
            off = sid * _RPS + j * _ZR
            pltpu.sync_copy(acc.at[pl.ds(off, _ZR)], out_hbm.at[cid, pl.ds(off, _ZR)])

    return conv_kernel(table, row3, col3, ew3, zeros)


def _dinv_block(d0_ref, d1_ref):
    deg = d0_ref[:, 0:1] + d1_ref[:, 0:1] + 1.0
    return lax.rsqrt(deg)


def _tc_stage1(x, d0, d1, w1):
    def body(x_ref, d0_ref, d1_ref, w_ref, o_ref):
        dinv = _dinv_block(d0_ref, d1_ref)
        xw = jnp.dot(x_ref[...], w_ref[...], preferred_element_type=jnp.float32)
        o_ref[...] = xw * dinv

    return pl.pallas_call(
        body,
        grid=(_NBLK,),
        in_specs=[
            pl.BlockSpec((_BLK, 128), lambda i: (i, 0)),
            pl.BlockSpec((_BLK, 128), lambda i: (i, 0)),
            pl.BlockSpec((_BLK, 128), lambda i: (i, 0)),
            pl.BlockSpec((128, 128), lambda i: (0, 0)),
        ],
        out_specs=pl.BlockSpec((_BLK, 128), lambda i: (i, 0)),
        out_shape=jax.ShapeDtypeStruct((_N, 128), jnp.float32),
    )(x, d0, d1, w1)


def _tc_stage2(a0, a1, t1, d0, d1, b1, w2):
    def body(a0_ref, a1_ref, t1_ref, d0_ref, d1_ref, b_ref, w_ref, o_ref):
        dinv = _dinv_block(d0_ref, d1_ref)
        h = dinv * (a0_ref[...] + a1_ref[...] + t1_ref[...]) + b_ref[...]
        h = jnp.maximum(h, 0.0)
        hw = jnp.dot(h, w_ref[...], preferred_element_type=jnp.float32)
        o_ref[...] = hw * dinv

    return pl.pallas_call(
        body,
        grid=(_NBLK,),
        in_specs=[
            pl.BlockSpec((_BLK, 128), lambda i: (i, 0)),
            pl.BlockSpec((_BLK, 128), lambda i: (i, 0)),
            pl.BlockSpec((_BLK, 128), lambda i: (i, 0)),
            pl.BlockSpec((_BLK, 128), lambda i: (i, 0)),
            pl.BlockSpec((_BLK, 128), lambda i: (i, 0)),
            pl.BlockSpec((1, 128), lambda i: (0, 0)),
            pl.BlockSpec((128, 128), lambda i: (0, 0)),
        ],
        out_specs=pl.BlockSpec((_BLK, 128), lambda i: (i, 0)),
        out_shape=jax.ShapeDtypeStruct((_N, 128), jnp.float32),
    )(a0, a1, t1, d0, d1, b1, w2)


def _tc_final(a0, a1, t2, d0, d1, b2, batch3, wl1, bl1, wl2, bl2):
    def body(a0_ref, a1_ref, t2_ref, d0_ref, d1_ref, b_ref, batch_ref,
             wl1_ref, bl1_ref, wl2_ref, bl2_ref,
             yp_ref, yh_ref, out_ref, pooled_ref):
        i = pl.program_id(0)

        @pl.when(i == 0)
        def _():
            pooled_ref[...] = jnp.zeros_like(pooled_ref)

        dinv = _dinv_block(d0_ref, d1_ref)
        h2 = dinv * (a0_ref[...] + a1_ref[...] + t2_ref[...]) + b_ref[...]
        h2 = jnp.maximum(h2, 0.0)
        b = batch_ref[...][0]
        gids = lax.broadcasted_iota(jnp.int32, (_G, _BLK), 0)
        onehot = (jnp.broadcast_to(b, (_G, _BLK)) == gids).astype(jnp.float32)
        pooled_ref[...] += jnp.dot(onehot, h2, preferred_element_type=jnp.float32)

        @pl.when(i == _NBLK - 1)
        def _():
            p = pooled_ref[...]
            h3 = jnp.dot(p, wl1_ref[...], preferred_element_type=jnp.float32)
            h3 = jnp.maximum(h3 + bl1_ref[...], 0.0)
            o = jnp.dot(h3, wl2_ref[...], preferred_element_type=jnp.float32)
            o = o + bl2_ref[...]
            m = jnp.max(o, axis=1, keepdims=True)
            lse = m + jnp.log(jnp.sum(jnp.exp(o - m), axis=1, keepdims=True))
            yp_ref[...] = o - lse
            yh_ref[...] = (o[:, 1:2] > o[:, 0:1]).astype(jnp.float32)
            out_ref[...] = o

    return pl.pallas_call(
        body,
        grid=(_NBLK,),
        in_specs=[
            pl.BlockSpec((_BLK, 128), lambda i: (i, 0)),
            pl.BlockSpec((_BLK, 128), lambda i: (i, 0)),
            pl.BlockSpec((_BLK, 128), lambda i: (i, 0)),
            pl.BlockSpec((_BLK, 128), lambda i: (i, 0)),
            pl.BlockSpec((_BLK, 128), lambda i: (i, 0)),
            pl.BlockSpec((1, 128), lambda i: (0, 0)),
            pl.BlockSpec((1, 1, _BLK), lambda i: (i, 0, 0)),
            pl.BlockSpec((128, 64), lambda i: (0, 0)),
            pl.BlockSpec((1, 64), lambda i: (0, 0)),
            pl.BlockSpec((64, 2), lambda i: (0, 0)),
            pl.BlockSpec((1, 2), lambda i: (0, 0)),
        ],
        out_specs=[
            pl.BlockSpec((_G, 2), lambda i: (0, 0)),
            pl.BlockSpec((_G, 1), lambda i: (0, 0)),
            pl.BlockSpec((_G, 2), lambda i: (0, 0)),
        ],
        out_shape=[
            jax.ShapeDtypeStruct((_G, 2), jnp.float32),
            jax.ShapeDtypeStruct((_G, 1), jnp.float32),
            jax.ShapeDtypeStruct((_G, 2), jnp.float32),
        ],
        scratch_shapes=[pltpu.VMEM((_G, 128), jnp.float32)],
    )(a0, a1, t2, d0, d1, b2, batch3, wl1, bl1, wl2, bl2)


def kernel(x, edge_index, edge_weight, batch, W1, b1, W2, b2, Wl1, bl1, Wl2, bl2):
    row = edge_index[0]
    col = edge_index[1]

    zeros = jnp.zeros((_ZR, 128), jnp.float32)
    deg16 = _sc_degree(col, edge_weight, zeros)
    d0 = deg16[0]
    d1 = deg16[1]

    w2p = jnp.pad(W2, ((0, 0), (0, 64)))
    b2p = jnp.pad(b2, (0, 64)).reshape(1, 128)
    wl1p = jnp.pad(Wl1, ((0, 64), (0, 0)))

    pad = _EPWP - _EPW
    row3 = jnp.pad(row.reshape(_NW, _EPW), ((0, 0), (0, pad))).reshape(
        _NW, _CNCH, 1, _CCH)
    col3 = jnp.pad(col.reshape(_NW, _EPW), ((0, 0), (0, pad))).reshape(
        _NW, _CNCH, 1, _CCH)
    ew3 = jnp.pad(edge_weight.reshape(_NW, _EPW), ((0, 0), (0, pad))).reshape(
        _NW, _CNCH, 1, _CCH)

    table1 = _tc_stage1(x, d0, d1, W1)
    agg1 = _sc_conv(table1, row3, col3, ew3, zeros, 128)
    table2 = _tc_stage2(agg1[0], agg1[1], table1, d0, d1,
                        b1.reshape(1, 128), w2p)
    agg2 = _sc_conv(table2, row3, col3, ew3, zeros, 128)

    batch3 = batch.reshape(_NBLK, 1, _BLK)
    y_prob, y_hat, out = _tc_final(agg2[0], agg2[1], table2, d0, d1,
                                   b2p, batch3,
                                   wl1p, bl1.reshape(1, 64),
                                   Wl2, bl2.reshape(1, 2))
    return (y_prob, y_hat.reshape(_G), out)


# speedup vs baseline: 1.1913x; 1.1295x over previous
import dataclasses
import functools

import jax
import jax.numpy as jnp
from jax import lax
from jax.experimental import pallas as pl
from jax.experimental.pallas import tpu as pltpu
from jax.experimental.pallas import tpu_sc as plsc

_N = 10000
_E = 320000
_G = 64
_NC = 2
_NS = 16
_NW = _NC * _NS
_EPW = _E // _NW
_CH = 80
_NCHUNK = _EPW // _CH
_NPAD = 10240
_RPS = _NPAD // _NS
_ZR = 128
_BLK = 400
_NBLK = _N // _BLK


def _vector_mesh():
    return plsc.VectorSubcoreMesh(core_axis_name="c", subcore_axis_name="s")


def _sc_params():
    cp = pltpu.CompilerParams()
    if "needs_layout_passes" in pltpu.CompilerParams.__dataclass_fields__:
        cp = dataclasses.replace(cp, needs_layout_passes=False)
    return cp


def _sc_degree(col3, ew3, zeros):
    @functools.partial(
        pl.kernel,
        out_type=jax.ShapeDtypeStruct((_NC, _NPAD, 128), jnp.float32),
        mesh=_vector_mesh(),
        compiler_params=_sc_params(),
        scratch_types=[
            pltpu.VMEM((4, 1, 128), jnp.int32),
            pltpu.VMEM((4, 1, 128), jnp.float32),
            pltpu.VMEM((128, 128), jnp.float32),
            pltpu.VMEM((128, 128), jnp.float32),
            pltpu.VMEM_SHARED((_NPAD, 128), jnp.float32),
            pltpu.SemaphoreType.DMA,
            pltpu.SemaphoreType.DMA,
            pltpu.SemaphoreType.DMA,
            pltpu.SemaphoreType.DMA,
            pltpu.SemaphoreType.DMA,
            pltpu.SemaphoreType.DMA,
        ],
    )
    def deg_kernel(col_hbm, ew_hbm, zero_hbm, out_hbm,
                   colq, ewq, src0, src1, acc, s0, s1, i0, i1, i2, i3):
        cid = lax.axis_index("c")
        sid = lax.axis_index("s")
        wid = sid * _NC + cid
        zero16 = jnp.zeros((16,), jnp.float32)
        srcs = (src0, src1)
        ssems = (s0, s1)
        isems = (i0, i1, i2, i3)

        @pl.loop(0, 128)
        def _(r):
            for kk in range(8):
                src0[r, pl.ds(kk * 16, 16)] = zero16
                src1[r, pl.ds(kk * 16, 16)] = zero16

        for j in range(_RPS // _ZR):
            pltpu.sync_copy(zero_hbm, acc.at[pl.ds(sid * _RPS + j * _ZR, _ZR)])
        plsc.subcore_barrier()

        def idx_start(ch, q):
            pltpu.async_copy(col_hbm.at[wid, ch], colq.at[q], isems[q])
            pltpu.async_copy(ew_hbm.at[wid, ch], ewq.at[q], isems[q])

        def idx_wait(q):
            pltpu.make_async_copy(col_hbm.at[wid, 0], colq.at[q], isems[q]).wait()
            pltpu.make_async_copy(ew_hbm.at[wid, 0], ewq.at[q], isems[q]).wait()

        def stage(b, q):
            zi = jnp.zeros((16,), jnp.int32)

            @plsc.parallel_loop(0, _CCH, unroll=4)
            def _(e):
                w = plsc.load_gather(ewq, [zi + q, zi, zi + e])
                srcs[b][e, pl.ds(0, 16)] = w

        def scatter_start(b, q):
            pltpu.async_copy(srcs[b], acc.at[colq.at[q, 0]], ssems[b], add=True)

        def scatter_wait(b):
            pltpu.make_async_copy(srcs[b], acc.at[colq.at[0, 0]], ssems[b]).wait()

        for q in range(4):
            idx_start(q, q)
        idx_wait(0)
        stage(0, 0)
        scatter_start(0, 0)
        idx_wait(1)
        stage(1, 1)
        scatter_start(1, 1)

        @pl.loop(0, _CNCH - 4, step=4)
        def _(p):
            scatter_wait(0)
            idx_start(p + 4, 0)
            idx_wait(2)
            stage(0, 2)
            scatter_start(0, 2)
            scatter_wait(1)
            idx_start(p + 5, 1)
            idx_wait(3)
            stage(1, 3)
            scatter_start(1, 3)
            scatter_wait(0)
            idx_start(p + 6, 2)
            idx_wait(0)
            stage(0, 0)
            scatter_start(0, 0)
            scatter_wait(1)
            idx_start(p + 7, 3)
            idx_wait(1)
            stage(1, 1)
            scatter_start(1, 1)

        scatter_wait(0)
        idx_wait(2)
        stage(0, 2)
        scatter_start(0, 2)
        scatter_wait(1)
        idx_wait(3)
        stage(1, 3)
        scatter_start(1, 3)
        scatter_wait(0)
        scatter_wait(1)

        plsc.subcore_barrier()
        for j in range(_RPS // _ZR):
            off = sid * _RPS + j * _ZR
            pltpu.sync_copy(acc.at[pl.ds(off, _ZR)], out_hbm.at[cid, pl.ds(off, _ZR)])

    return deg_kernel(col3, ew3, zeros)


_EPWP = 10240
_CCH = 128
_CNCH = _EPWP // _CCH


def _sc_conv(table, row3, col3, ew3, zeros, d):
    @functools.partial(
        pl.kernel,
        out_type=jax.ShapeDtypeStruct((_NC, _NPAD, d), jnp.float32),
        mesh=_vector_mesh(),
        compiler_params=_sc_params(),
        scratch_types=[
            pltpu.VMEM((4, 1, _CCH), jnp.int32),
            pltpu.VMEM((4, 1, _CCH), jnp.int32),
            pltpu.VMEM((4, 1, _CCH), jnp.float32),
            pltpu.VMEM((_CCH, d), jnp.float32),
            pltpu.VMEM((_CCH, d), jnp.float32),
            pltpu.VMEM_SHARED((_NPAD, d), jnp.float32),
            pltpu.SemaphoreType.DMA,
            pltpu.SemaphoreType.DMA,
            pltpu.SemaphoreType.DMA,
            pltpu.SemaphoreType.DMA,
            pltpu.SemaphoreType.DMA,
            pltpu.SemaphoreType.DMA,
            pltpu.SemaphoreType.DMA,
            pltpu.SemaphoreType.DMA,
        ],
    )
    def conv_kernel(table_hbm, row_hbm, col_hbm, ew_hbm, zero_hbm, out_hbm,
                    rowq, colq, ewq, msg0, msg1, acc,
                    g0, g1, s0, s1, i0, i1, i2, i3):
        cid = lax.axis_index("c")
        sid = lax.axis_index("s")
        wid = sid * _NC + cid
        msgs = (msg0, msg1)
        gsems = (g0, g1)
        ssems = (s0, s1)
        isems = (i0, i1, i2, i3)

        for j in range(_RPS // _ZR):
            pltpu.sync_copy(zero_hbm, acc.at[pl.ds(sid * _RPS + j * _ZR, _ZR)])
        plsc.subcore_barrier()

        def idx_start(ch, q):
            pltpu.async_copy(row_hbm.at[wid, ch], rowq.at[q], isems[q])
            pltpu.async_copy(col_hbm.at[wid, ch], colq.at[q], isems[q])
            pltpu.async_copy(ew_hbm.at[wid, ch], ewq.at[q], isems[q])

        def idx_wait(q):
            pltpu.make_async_copy(row_hbm.at[wid, 0], rowq.at[q], isems[q]).wait()
            pltpu.make_async_copy(col_hbm.at[wid, 0], colq.at[q], isems[q]).wait()
            pltpu.make_async_copy(ew_hbm.at[wid, 0], ewq.at[q], isems[q]).wait()

        def scale(b, q):
            zi = jnp.zeros((16,), jnp.int32)

            @plsc.parallel_loop(0, _CCH, unroll=4)
            def _(e):
                w = plsc.load_gather(ewq, [zi + q, zi, zi + e])
                for kk in range(d // 16):
                    msgs[b][e, pl.ds(kk * 16, 16)] = (
                        msgs[b][e, pl.ds(kk * 16, 16)] * w)

        def gather_start(b, q):
            pltpu.async_copy(table_hbm.at[rowq.at[q, 0]], msgs[b], gsems[b])

        def gather_wait(b):
            pltpu.make_async_copy(table_hbm.at[rowq.at[0, 0]], msgs[b],
                                  gsems[b]).wait()

        def scatter_start(b, q):
            pltpu.async_copy(msgs[b], acc.at[colq.at[q, 0]], ssems[b], add=True)

        def scatter_wait(b):
            pltpu.make_async_copy(msgs[b], acc.at[colq.at[0, 0]], ssems[b]).wait()

        for q in range(4):
            idx_start(q, q)
        idx_wait(0)
        gather_start(0, 0)
        idx_wait(1)
        gather_start(1, 1)

        @pl.loop(0, _CNCH - 4, step=4)
        def _(p):
            gather_wait(0)
            scale(0, 0)
            scatter_start(0, 0)
            gather_wait(1)
            scale(1, 1)
            scatter_start(1, 1)
            scatter_wait(0)
            idx_start(p + 4, 0)
            idx_wait(2)
            gather_start(0, 2)
            scatter_wait(1)
            idx_start(p + 5, 1)
            idx_wait(3)
            gather_start(1, 3)
            gather_wait(0)
            scale(0, 2)
            scatter_start(0, 2)
            gather_wait(1)
            scale(1, 3)
            scatter_start(1, 3)
            scatter_wait(0)
            idx_start(p + 6, 2)
            idx_wait(0)
            gather_start(0, 0)
            scatter_wait(1)
            idx_start(p + 7, 3)
            idx_wait(1)
            gather_start(1, 1)

        gather_wait(0)
        scale(0, 0)
        scatter_start(0, 0)
        gather_wait(1)
        scale(1, 1)
        scatter_start(1, 1)
        scatter_wait(0)
        idx_wait(2)
        gather_start(0, 2)
        scatter_wait(1)
        idx_wait(3)
        gather_start(1, 3)
        gather_wait(0)
        scale(0, 2)
        scatter_start(0, 2)
        gather_wait(1)
        scale(1, 3)
        scatter_start(1, 3)
        scatter_wait(0)
        scatter_wait(1)

        plsc.subcore_barrier()
        for j in range(_RPS // _ZR):
            off = sid * _RPS + j * _ZR
            pltpu.sync_copy(acc.at[pl.ds(off, _ZR)], out_hbm.at[cid, pl.ds(off, _ZR)])

    return conv_kernel(table, row3, col3, ew3, zeros)


def _dinv_block(d0_ref, d1_ref):
    deg = d0_ref[:, 0:1] + d1_ref[:, 0:1] + 1.0
    return lax.rsqrt(deg)


def _tc_stage1(x, d0, d1, w1):
    def body(x_ref, d0_ref, d1_ref, w_ref, o_ref):
        dinv = _dinv_block(d0_ref, d1_ref)
        xw = jnp.dot(x_ref[...], w_ref[...], preferred_element_type=jnp.float32)
        o_ref[...] = xw * dinv

    return pl.pallas_call(
        body,
        grid=(_NBLK,),
        in_specs=[
            pl.BlockSpec((_BLK, 128), lambda i: (i, 0)),
            pl.BlockSpec((_BLK, 128), lambda i: (i, 0)),
            pl.BlockSpec((_BLK, 128), lambda i: (i, 0)),
            pl.BlockSpec((128, 128), lambda i: (0, 0)),
        ],
        out_specs=pl.BlockSpec((_BLK, 128), lambda i: (i, 0)),
        out_shape=jax.ShapeDtypeStruct((_N, 128), jnp.float32),
    )(x, d0, d1, w1)


def _tc_stage2(a0, a1, t1, d0, d1, b1, w2):
    def body(a0_ref, a1_ref, t1_ref, d0_ref, d1_ref, b_ref, w_ref, o_ref):
        dinv = _dinv_block(d0_ref, d1_ref)
        h = dinv * (a0_ref[...] + a1_ref[...] + t1_ref[...]) + b_ref[...]
        h = jnp.maximum(h, 0.0)
        hw = jnp.dot(h, w_ref[...], preferred_element_type=jnp.float32)
        o_ref[...] = hw * dinv

    return pl.pallas_call(
        body,
        grid=(_NBLK,),
        in_specs=[
            pl.BlockSpec((_BLK, 128), lambda i: (i, 0)),
            pl.BlockSpec((_BLK, 128), lambda i: (i, 0)),
            pl.BlockSpec((_BLK, 128), lambda i: (i, 0)),
            pl.BlockSpec((_BLK, 128), lambda i: (i, 0)),
            pl.BlockSpec((_BLK, 128), lambda i: (i, 0)),
            pl.BlockSpec((1, 128), lambda i: (0, 0)),
            pl.BlockSpec((128, 128), lambda i: (0, 0)),
        ],
        out_specs=pl.BlockSpec((_BLK, 128), lambda i: (i, 0)),
        out_shape=jax.ShapeDtypeStruct((_N, 128), jnp.float32),
    )(a0, a1, t1, d0, d1, b1, w2)


def _tc_final(a0, a1, t2, d0, d1, b2, batch3, wl1, bl1, wl2, bl2):
    def body(a0_ref, a1_ref, t2_ref, d0_ref, d1_ref, b_ref, batch_ref,
             wl1_ref, bl1_ref, wl2_ref, bl2_ref,
             yp_ref, yh_ref, out_ref, pooled_ref):
        i = pl.program_id(0)

        @pl.when(i == 0)
        def _():
            pooled_ref[...] = jnp.zeros_like(pooled_ref)

        dinv = _dinv_block(d0_ref, d1_ref)
        h2 = dinv * (a0_ref[...] + a1_ref[...] + t2_ref[...]) + b_ref[...]
        h2 = jnp.maximum(h2, 0.0)
        b = batch_ref[...][0]
        gids = lax.broadcasted_iota(jnp.int32, (_G, _BLK), 0)
        onehot = (jnp.broadcast_to(b, (_G, _BLK)) == gids).astype(jnp.float32)
        pooled_ref[...] += jnp.dot(onehot, h2, preferred_element_type=jnp.float32)

        @pl.when(i == _NBLK - 1)
        def _():
            p = pooled_ref[...]
            h3 = jnp.dot(p, wl1_ref[...], preferred_element_type=jnp.float32)
            h3 = jnp.maximum(h3 + bl1_ref[...], 0.0)
            o = jnp.dot(h3, wl2_ref[...], preferred_element_type=jnp.float32)
            o = o + bl2_ref[...]
            m = jnp.max(o, axis=1, keepdims=True)
            lse = m + jnp.log(jnp.sum(jnp.exp(o - m), axis=1, keepdims=True))
            yp_ref[...] = o - lse
            yh_ref[...] = (o[:, 1:2] > o[:, 0:1]).astype(jnp.float32)
            out_ref[...] = o

    return pl.pallas_call(
        body,
        grid=(_NBLK,),
        in_specs=[
            pl.BlockSpec((_BLK, 128), lambda i: (i, 0)),
            pl.BlockSpec((_BLK, 128), lambda i: (i, 0)),
            pl.BlockSpec((_BLK, 128), lambda i: (i, 0)),
            pl.BlockSpec((_BLK, 128), lambda i: (i, 0)),
            pl.BlockSpec((_BLK, 128), lambda i: (i, 0)),
            pl.BlockSpec((1, 128), lambda i: (0, 0)),
            pl.BlockSpec((1, 1, _BLK), lambda i: (i, 0, 0)),
            pl.BlockSpec((128, 64), lambda i: (0, 0)),
            pl.BlockSpec((1, 64), lambda i: (0, 0)),
            pl.BlockSpec((64, 2), lambda i: (0, 0)),
            pl.BlockSpec((1, 2), lambda i: (0, 0)),
        ],
        out_specs=[
            pl.BlockSpec((_G, 2), lambda i: (0, 0)),
            pl.BlockSpec((_G, 1), lambda i: (0, 0)),
            pl.BlockSpec((_G, 2), lambda i: (0, 0)),
        ],
        out_shape=[
            jax.ShapeDtypeStruct((_G, 2), jnp.float32),
            jax.ShapeDtypeStruct((_G, 1), jnp.float32),
            jax.ShapeDtypeStruct((_G, 2), jnp.float32),
        ],
        scratch_shapes=[pltpu.VMEM((_G, 128), jnp.float32)],
    )(a0, a1, t2, d0, d1, b2, batch3, wl1, bl1, wl2, bl2)


def kernel(x, edge_index, edge_weight, batch, W1, b1, W2, b2, Wl1, bl1, Wl2, bl2):
    row = edge_index[0]
    col = edge_index[1]

    zeros = jnp.zeros((_ZR, 128), jnp.float32)

    w2p = jnp.pad(W2, ((0, 0), (0, 64)))
    b2p = jnp.pad(b2, (0, 64)).reshape(1, 128)
    wl1p = jnp.pad(Wl1, ((0, 64), (0, 0)))

    pad = _EPWP - _EPW
    row3 = jnp.pad(row.reshape(_NW, _EPW), ((0, 0), (0, pad))).reshape(
        _NW, _CNCH, 1, _CCH)
    col3 = jnp.pad(col.reshape(_NW, _EPW), ((0, 0), (0, pad))).reshape(
        _NW, _CNCH, 1, _CCH)
    ew3 = jnp.pad(edge_weight.reshape(_NW, _EPW), ((0, 0), (0, pad))).reshape(
        _NW, _CNCH, 1, _CCH)

    deg16 = _sc_degree(col3, ew3, zeros)
    d0 = deg16[0]
    d1 = deg16[1]

    table1 = _tc_stage1(x, d0, d1, W1)
    agg1 = _sc_conv(table1, row3, col3, ew3, zeros, 128)
    table2 = _tc_stage2(agg1[0], agg1[1], table1, d0, d1,
                        b1.reshape(1, 128), w2p)
    agg2 = _sc_conv(table2, row3, col3, ew3, zeros, 128)

    batch3 = batch.reshape(_NBLK, 1, _BLK)
    y_prob, y_hat, out = _tc_final(agg2[0], agg2[1], table2, d0, d1,
                                   b2p, batch3,
                                   wl1p, bl1.reshape(1, 64),
                                   Wl2, bl2.reshape(1, 2))
    return (y_prob, y_hat.reshape(_G), out)
